# baseline (speedup 1.0000x reference)
import jax
import jax.numpy as jnp
from jax.experimental import pallas as pl


def _router_block(x_ref, wt_ref, b_ref, o_ref):
    logits = jnp.dot(x_ref[...], wt_ref[...],
                     preferred_element_type=jnp.float32)
    logits = logits + b_ref[...]
    m = jnp.max(logits, axis=-1, keepdims=True)
    e = jnp.exp(logits - m)
    o_ref[...] = e / jnp.sum(e, axis=-1, keepdims=True)


def kernel(x, W, b):
    n_tokens, d_model = x.shape
    n_experts = W.shape[0]
    block_t = 1024
    wt = W.T
    b2 = b.reshape(1, n_experts)
    return pl.pallas_call(
        _router_block,
        grid=(n_tokens // block_t,),
        in_specs=[
            pl.BlockSpec((block_t, d_model), lambda i: (i, 0)),
            pl.BlockSpec((d_model, n_experts), lambda i: (0, 0)),
            pl.BlockSpec((1, n_experts), lambda i: (0, 0)),
        ],
        out_specs=pl.BlockSpec((block_t, n_experts), lambda i: (i, 0)),
        out_shape=jax.ShapeDtypeStruct((n_tokens, n_experts), jnp.float32),
    )(x, wt, b2)


# speedup vs baseline: 1.0008x; 1.0008x over previous
import jax
import jax.numpy as jnp
from jax.experimental import pallas as pl


def _router_block(x_ref, wt_ref, b_ref, o_ref):
    logits = jnp.dot(x_ref[...].astype(jnp.bfloat16),
                     wt_ref[...].astype(jnp.bfloat16),
                     preferred_element_type=jnp.float32)
    logits = logits + b_ref[...]
    m = jnp.max(logits, axis=-1, keepdims=True)
    e = jnp.exp(logits - m)
    o_ref[...] = e / jnp.sum(e, axis=-1, keepdims=True)


def kernel(x, W, b):
    n_tokens, d_model = x.shape
    n_experts = W.shape[0]
    block_t = 1024
    wt = W.T
    b2 = b.reshape(1, n_experts)
    return pl.pallas_call(
        _router_block,
        grid=(n_tokens // block_t,),
        in_specs=[
            pl.BlockSpec((block_t, d_model), lambda i: (i, 0)),
            pl.BlockSpec((d_model, n_experts), lambda i: (0, 0)),
            pl.BlockSpec((1, n_experts), lambda i: (0, 0)),
        ],
        out_specs=pl.BlockSpec((block_t, n_experts), lambda i: (i, 0)),
        out_shape=jax.ShapeDtypeStruct((n_tokens, n_experts), jnp.float32),
    )(x, wt, b2)
